# Initial kernel scaffold; baseline (speedup 1.0000x reference)
import jax
import jax.numpy as jnp
from jax.experimental import pallas as pl


def kernel(x, edge_index, W1, b1, W2, b2, Ws, bs, Wd, bd):
    raise NotImplementedError("write your pallas kernel here")


# speedup vs baseline: 6.2079x; 6.2079x over previous
import functools
import math

import jax
import jax.numpy as jnp
from jax import lax
from jax.experimental import pallas as pl
from jax.experimental.pallas import tpu as pltpu
from jax.experimental.pallas import tpu_sc as plsc

N_NODES = 10000
N_EDGES = 320000
IN_DIM = 128
PROJ = 64
LANES = 16

NC = 2
NS = 16
NW = NC * NS
E_PER_W = N_EDGES // NW
CHUNK = 80
CHUNKS_PER_W = E_PER_W // CHUNK
ROWS_PER_TILE = N_NODES // NS


def _dense_body(x_ref, w1_ref, b1_ref, w2_ref, b2_ref, ws_ref, bs_ref,
                wd_ref, bd_ref, h_ref, hs_ref, hd_ref):
    x = x_ref[...]
    h1 = jnp.maximum(
        jnp.dot(x, w1_ref[...], preferred_element_type=jnp.float32)
        + b1_ref[...], 0.0)
    h = (jnp.dot(h1, w2_ref[...], preferred_element_type=jnp.float32)
         + b2_ref[...])
    h_ref[...] = h
    hs_ref[...] = (jnp.dot(h, ws_ref[...], preferred_element_type=jnp.float32)
                   + bs_ref[...])
    hd_ref[...] = (jnp.dot(h, wd_ref[...], preferred_element_type=jnp.float32)
                   + bd_ref[...])


def _dense(x, W1, b1, W2, b2, Ws, bs, Wd, bd):
    out_t = jax.ShapeDtypeStruct((N_NODES, PROJ), jnp.float32)
    return pl.pallas_call(
        _dense_body,
        out_shape=(out_t, out_t, out_t),
    )(x, W1, b1.reshape(1, -1), W2, b2.reshape(1, -1),
      Ws, bs.reshape(1, -1), Wd, bd.reshape(1, -1))


def _edge_body(hs_hbm, hd_hbm, src_hbm, dst_hbm, zeros_hbm, p_hbm,
               srcv, dstv, hsr, hdr, mr, agg, sem0, sem1):
    cid = lax.axis_index("c")
    sid = lax.axis_index("s")
    wid = cid * NS + sid

    r0 = sid * ROWS_PER_TILE
    pltpu.sync_copy(zeros_hbm.at[pl.ds(r0, ROWS_PER_TILE)],
                    agg.at[pl.ds(r0, ROWS_PER_TILE)])
    plsc.subcore_barrier()

    base = wid * CHUNKS_PER_W

    @pl.loop(0, CHUNKS_PER_W)
    def _(j):
        row = base + j
        pltpu.sync_copy(src_hbm.at[pl.ds(row, 1)], srcv)
        pltpu.sync_copy(dst_hbm.at[pl.ds(row, 1)], dstv)
        cp0 = pltpu.async_copy(hs_hbm.at[srcv.at[0]], hsr, sem0)
        cp1 = pltpu.async_copy(hd_hbm.at[dstv.at[0]], hdr, sem1)
        cp0.wait()
        cp1.wait()

        @pl.loop(0, CHUNK)
        def _(e):
            a = hsr[e, pl.ds(0, LANES)]
            b = hsr[e, pl.ds(LANES, LANES)]
            c = hsr[e, pl.ds(2 * LANES, LANES)]
            d = hsr[e, pl.ds(3 * LANES, LANES)]
            acc = (a * hdr[e, pl.ds(0, LANES)]
                   + b * hdr[e, pl.ds(LANES, LANES)]
                   + c * hdr[e, pl.ds(2 * LANES, LANES)]
                   + d * hdr[e, pl.ds(3 * LANES, LANES)])
            s = jnp.sum(acc) * (1.0 / math.sqrt(PROJ))
            sv = jnp.full((LANES,), s, jnp.float32)
            w = 1.0 / (1.0 + jnp.exp(-sv))
            mr[e, pl.ds(0, LANES)] = a * w
            mr[e, pl.ds(LANES, LANES)] = b * w
            mr[e, pl.ds(2 * LANES, LANES)] = c * w
            mr[e, pl.ds(3 * LANES, LANES)] = d * w

        pltpu.sync_copy(mr, agg.at[dstv.at[0]], add=True)

    plsc.subcore_barrier()
    pltpu.sync_copy(agg.at[pl.ds(r0, ROWS_PER_TILE)],
                    p_hbm.at[cid, pl.ds(r0, ROWS_PER_TILE)])


def _edge(hs, hd, src2d, dst2d, zeros):
    mesh = plsc.VectorSubcoreMesh(core_axis_name="c", subcore_axis_name="s")
    k = pl.kernel(
        _edge_body,
        out_type=jax.ShapeDtypeStruct((NC, N_NODES, PROJ), jnp.float32),
        mesh=mesh,
        compiler_params=pltpu.CompilerParams(
            use_tc_tiling_on_sc=False, needs_layout_passes=False),
        scratch_types=[
            pltpu.VMEM((1, CHUNK), jnp.int32),
            pltpu.VMEM((1, CHUNK), jnp.int32),
            pltpu.VMEM((CHUNK, PROJ), jnp.float32),
            pltpu.VMEM((CHUNK, PROJ), jnp.float32),
            pltpu.VMEM((CHUNK, PROJ), jnp.float32),
            pltpu.VMEM_SHARED((N_NODES, PROJ), jnp.float32),
            pltpu.SemaphoreType.DMA,
            pltpu.SemaphoreType.DMA,
        ],
    )
    return k(hs, hd, src2d, dst2d, zeros)


def _assemble_body(h_ref, p_ref, o_ref):
    o_ref[...] = jnp.concatenate(
        [h_ref[...], p_ref[0] + p_ref[1]], axis=-1)


def _assemble(h, p):
    return pl.pallas_call(
        _assemble_body,
        out_shape=jax.ShapeDtypeStruct((N_NODES, 2 * PROJ), jnp.float32),
    )(h, p)


def kernel(x, edge_index, W1, b1, W2, b2, Ws, bs, Wd, bd):
    h, hs, hd = _dense(x, W1, b1, W2, b2, Ws, bs, Wd, bd)
    src2d = edge_index[0].reshape(N_EDGES // CHUNK, CHUNK)
    dst2d = edge_index[1].reshape(N_EDGES // CHUNK, CHUNK)
    zeros = jnp.zeros((N_NODES, PROJ), jnp.float32)
    p = _edge(hs, hd, src2d, dst2d, zeros)
    return _assemble(h, p)
